# baseline (speedup 1.0000x reference)
import dataclasses
import functools

import jax
import jax.numpy as jnp
from jax import lax
from jax.experimental import pallas as pl
from jax.experimental.pallas import tpu as pltpu
from jax.experimental.pallas import tpu_sc as plsc

VOCAB = 1000000
EDIM = 64
BATCH = 16384
NC = 2
NS = 16
L = 16
NW = NC * NS
BPW = BATCH // NW


def _glove_sc(word_u, word_v, in_embed, in_bias_1d, out_embed, out_bias_1d):
    mesh = plsc.VectorSubcoreMesh(core_axis_name="c", subcore_axis_name="s")
    cp = pltpu.CompilerParams(use_tc_tiling_on_sc=False)
    if "needs_layout_passes" in pltpu.CompilerParams.__dataclass_fields__:
        cp = dataclasses.replace(cp, needs_layout_passes=False)

    @functools.partial(
        pl.kernel,
        mesh=mesh,
        compiler_params=cp,
        out_type=jax.ShapeDtypeStruct((BATCH,), jnp.float32),
        scratch_types=[
            pltpu.VMEM((BPW,), jnp.int32),
            pltpu.VMEM((BPW,), jnp.int32),
            pltpu.VMEM((BPW, EDIM), jnp.float32),
            pltpu.VMEM((BPW, EDIM), jnp.float32),
            pltpu.VMEM((BPW,), jnp.float32),
            pltpu.VMEM((BPW,), jnp.float32),
            pltpu.VMEM((BPW + L,), jnp.float32),
            pltpu.SemaphoreType.DMA,
            pltpu.SemaphoreType.DMA,
            pltpu.SemaphoreType.DMA,
            pltpu.SemaphoreType.DMA,
        ],
    )
    def k(u_hbm, v_hbm, ie_hbm, ib_hbm, oe_hbm, ob_hbm, out_hbm,
          idx_u, idx_v, u_rows, v_rows, ub, vb, acc,
          sem_u, sem_v, sem_ub, sem_vb):
        wid = lax.axis_index("s") * NC + lax.axis_index("c")
        base = wid * BPW
        pltpu.sync_copy(u_hbm.at[pl.ds(base, BPW)], idx_u)
        pltpu.sync_copy(v_hbm.at[pl.ds(base, BPW)], idx_v)
        cu = pltpu.async_copy(ie_hbm.at[idx_u], u_rows, sem_u)
        cv = pltpu.async_copy(oe_hbm.at[idx_v], v_rows, sem_v)
        cub = pltpu.async_copy(ib_hbm.at[idx_u], ub, sem_ub)
        cvb = pltpu.async_copy(ob_hbm.at[idx_v], vb, sem_vb)
        cu.wait()
        cv.wait()

        last_lane = jnp.arange(L, dtype=jnp.int32) == (L - 1)

        @pl.loop(0, BPW)
        def _(r):
            a0 = u_rows[r, pl.ds(0, L)] * v_rows[r, pl.ds(0, L)]
            a1 = u_rows[r, pl.ds(L, L)] * v_rows[r, pl.ds(L, L)]
            a2 = u_rows[r, pl.ds(2 * L, L)] * v_rows[r, pl.ds(2 * L, L)]
            a3 = u_rows[r, pl.ds(3 * L, L)] * v_rows[r, pl.ds(3 * L, L)]
            s = plsc.cumsum((a0 + a1) + (a2 + a3))
            plsc.store_compressed(acc.at[pl.ds(r, L)], s, mask=last_lane)

        cub.wait()
        cvb.wait()

        @pl.loop(0, BPW, step=L)
        def _(g):
            s = pl.ds(g, L)
            acc[s] = acc[s] + ub[s] + vb[s]

        pltpu.sync_copy(acc.at[pl.ds(0, BPW)], out_hbm.at[pl.ds(base, BPW)])

    return k(word_u, word_v, in_embed, in_bias_1d, out_embed, out_bias_1d)


def kernel(word_u, word_v, in_embed, in_bias, out_embed, out_bias):
    word_u = word_u.astype(jnp.int32)
    word_v = word_v.astype(jnp.int32)
    return _glove_sc(word_u, word_v, in_embed,
                     in_bias.reshape(VOCAB), out_embed,
                     out_bias.reshape(VOCAB))
